# baseline (speedup 1.0000x reference)
import functools

import jax
import jax.numpy as jnp
from jax import lax
from jax.experimental import pallas as pl
from jax.experimental.pallas import tpu as pltpu
from jax.experimental.pallas import tpu_sc as plsc


def _gate_body(nsteps, E, S, x_ref, w_ref, b_ref, idx_ref, val_ref, imp_ref,
               acc_ref):
    i = pl.program_id(0)
    xb = x_ref[...]
    logits = jnp.dot(xb, w_ref[...], preferred_element_type=jnp.float32)
    logits = logits + b_ref[...]

    ii = jax.lax.broadcasted_iota(jnp.int32, logits.shape, 1)
    m = jnp.max(logits, axis=-1, keepdims=True)
    idx1 = jnp.min(jnp.where(logits == m, ii, E), axis=-1, keepdims=True)
    masked = jnp.where(ii == idx1, -jnp.inf, logits)
    v2l = jnp.max(masked, axis=-1, keepdims=True)
    idx2 = jnp.min(jnp.where(masked == v2l, ii, E), axis=-1, keepdims=True)

    e = jnp.exp(logits - m)
    rcp_s = 1.0 / jnp.sum(e, axis=-1, keepdims=True)
    v1 = rcp_s
    v2 = jnp.exp(v2l - m) * rcp_s

    idx_ref[...] = jnp.transpose(jnp.concatenate([idx1, idx2], axis=1))
    val_ref[...] = jnp.transpose(jnp.concatenate([v1, v2], axis=1))

    @pl.when(i == 0)
    def _():
        acc_ref[...] = jnp.zeros_like(acc_ref)

    acc_ref[...] += jnp.sum(e * rcp_s, axis=0, keepdims=True)

    @pl.when(i == nsteps - 1)
    def _():
        imp_ref[...] = acc_ref[...]


def _tc_gate(x, W, b):
    S, D = x.shape
    E = W.shape[1]
    BS = 1024
    nsteps = S // BS
    b2 = b.reshape(1, E)

    body = functools.partial(_gate_body, nsteps, E, S)
    return pl.pallas_call(
        body,
        grid=(nsteps,),
        in_specs=[
            pl.BlockSpec((BS, D), lambda i: (i, 0)),
            pl.BlockSpec((D, E), lambda i: (0, 0)),
            pl.BlockSpec((1, E), lambda i: (0, 0)),
        ],
        out_specs=[
            pl.BlockSpec((2, BS), lambda i: (0, i)),
            pl.BlockSpec((2, BS), lambda i: (0, i)),
            pl.BlockSpec((1, E), lambda i: (0, 0)),
        ],
        out_shape=[
            jax.ShapeDtypeStruct((2, S), jnp.int32),
            jax.ShapeDtypeStruct((2, S), jnp.float32),
            jax.ShapeDtypeStruct((1, E), jnp.float32),
        ],
        scratch_shapes=[pltpu.VMEM((1, E), jnp.float32)],
        compiler_params=pltpu.CompilerParams(
            vmem_limit_bytes=60 * 1024 * 1024),
    )(x, W, b2)


def _sc_aux(idx2s, E, S):
    info = plsc.get_sparse_core_info()
    NC, NS, L = info.num_cores, info.num_subcores, info.num_lanes
    NW = NC * NS
    per_w = S // NW
    k = per_w // 128
    mesh = plsc.VectorSubcoreMesh(core_axis_name="c", subcore_axis_name="s")

    @functools.partial(
        pl.kernel, mesh=mesh,
        out_type=jax.ShapeDtypeStruct((NW * E,), jnp.float32),
        scratch_types=[
            pltpu.VMEM((k, 128), jnp.int32),
            pltpu.VMEM((128,), jnp.float32),
            pltpu.VMEM((E,), jnp.float32),
            pltpu.VMEM_SHARED((NS * E,), jnp.float32),
        ],
    )
    def sc_kernel(idx_hbm, bins_hbm, idxb, onesb, binsb, sbins):
        cid = lax.axis_index("c")
        sid = lax.axis_index("s")
        wid = sid * NC + cid

        for j in range(128 // L):
            onesb[pl.ds(j * L, L)] = jnp.ones((L,), jnp.float32)
        for j in range(E // L):
            binsb[pl.ds(j * L, L)] = jnp.zeros((L,), jnp.float32)
        pltpu.sync_copy(binsb, sbins.at[pl.ds(sid * E, E)])

        base = wid * per_w
        for j in range(k):
            pltpu.sync_copy(idx_hbm.at[pl.ds(base + j * 128, 128)],
                            idxb.at[j])
        off = sid * E
        for j in range(k):
            for m in range(128 // L):
                s = pl.ds(m * L, L)
                idxb[j, s] = idxb[j, s] + off

        for j in range(k):
            pltpu.sync_copy(onesb, sbins.at[idxb.at[j]], add=True)

        pltpu.sync_copy(sbins.at[pl.ds(sid * E, E)], binsb)
        pltpu.sync_copy(binsb, bins_hbm.at[pl.ds(wid * E, E)])

    return sc_kernel(idx2s)


def kernel(x, W, b):
    S, D = x.shape
    E = W.shape[1]
    idx_out, val_out, imp_out = _tc_gate(x, W, b)
    bins_parts = _sc_aux(idx_out[0], E, S)
    load_counts = jnp.sum(bins_parts.reshape(-1, E), axis=0)
    aux = (E / (S * S)) * jnp.dot(imp_out.reshape(E), load_counts)
    return (idx_out.T, val_out.T, aux)


# speedup vs baseline: 1.1601x; 1.1601x over previous
import functools

import jax
import jax.numpy as jnp
from jax.experimental import pallas as pl
from jax.experimental.pallas import tpu as pltpu


def _gate_body(nsteps, E, S, x_ref, w_ref, b_ref, idx_ref, val_ref, aux_ref,
               acc_ref):
    i = pl.program_id(0)
    xb = x_ref[...]
    logits = jnp.dot(xb, w_ref[...], preferred_element_type=jnp.float32)
    logits = logits + b_ref[...]

    ii = jax.lax.broadcasted_iota(jnp.int32, logits.shape, 1)
    m = jnp.max(logits, axis=-1, keepdims=True)
    idx1 = jnp.min(jnp.where(logits == m, ii, E), axis=-1, keepdims=True)
    masked = jnp.where(ii == idx1, -jnp.inf, logits)
    v2l = jnp.max(masked, axis=-1, keepdims=True)
    idx2 = jnp.min(jnp.where(masked == v2l, ii, E), axis=-1, keepdims=True)

    e = jnp.exp(logits - m)
    rcp_s = 1.0 / jnp.sum(e, axis=-1, keepdims=True)
    probs = e * rcp_s
    v1 = rcp_s
    v2 = jnp.exp(v2l - m) * rcp_s

    idx_ref[...] = jnp.transpose(jnp.concatenate([idx1, idx2], axis=1))
    val_ref[...] = jnp.transpose(jnp.concatenate([v1, v2], axis=1))

    @pl.when(i == 0)
    def _():
        acc_ref[...] = jnp.zeros_like(acc_ref)

    onehot = (ii == idx1).astype(jnp.float32)
    part = jnp.stack(
        [jnp.sum(probs, axis=0), jnp.sum(onehot, axis=0)], axis=0)
    acc_ref[...] += part

    @pl.when(i == nsteps - 1)
    def _():
        acc = acc_ref[...]
        aux_ref[0, 0] = (E / (S * S)) * jnp.sum(acc[0:1, :] * acc[1:2, :])


def kernel(x, W, b):
    S, D = x.shape
    E = W.shape[1]
    BS = 1024
    nsteps = S // BS
    b2 = b.reshape(1, E)

    body = functools.partial(_gate_body, nsteps, E, S)
    idx_out, val_out, aux_out = pl.pallas_call(
        body,
        grid=(nsteps,),
        in_specs=[
            pl.BlockSpec((BS, D), lambda i: (i, 0)),
            pl.BlockSpec((D, E), lambda i: (0, 0)),
            pl.BlockSpec((1, E), lambda i: (0, 0)),
        ],
        out_specs=[
            pl.BlockSpec((2, BS), lambda i: (0, i)),
            pl.BlockSpec((2, BS), lambda i: (0, i)),
            pl.BlockSpec((1, 1), lambda i: (0, 0), memory_space=pltpu.SMEM),
        ],
        out_shape=[
            jax.ShapeDtypeStruct((2, S), jnp.int32),
            jax.ShapeDtypeStruct((2, S), jnp.float32),
            jax.ShapeDtypeStruct((1, 1), jnp.float32),
        ],
        scratch_shapes=[pltpu.VMEM((2, E), jnp.float32)],
        compiler_params=pltpu.CompilerParams(
            vmem_limit_bytes=60 * 1024 * 1024),
    )(x, W, b2)
    return (idx_out.T, val_out.T, aux_out[0, 0])
